# baseline (speedup 1.0000x reference)
import functools

import jax
import jax.numpy as jnp
from jax import lax
from jax.experimental import pallas as pl
from jax.experimental.pallas import tpu as pltpu
from jax.experimental.pallas import tpu_sc as plsc

BATCH = 4096
SEQ = 200
EMBED = 64
VOCAB = 1000000
NUM_CORES = 2
NUM_SUBCORES = 16
NUM_WORKERS = NUM_CORES * NUM_SUBCORES
BPW = BATCH // NUM_WORKERS
LANES = 16
PAIR = 2 * EMBED
NBLK = SEQ // 8


def _body(idxT_hbm, tok2_hbm, posT_hbm, outT_hbm, idx8_v, pair_a, pair_b,
          rows_a, rows_b, outv_a, outv_b, posT_v, pos_v, sem_i, sg0, sg1,
          so0, so1):
    wid = lax.axis_index("s") * NUM_CORES + lax.axis_index("c")
    b0 = wid * BPW
    pairs = [pair_a, pair_b]
    rows = [rows_a, rows_b]
    outs = [outv_a, outv_b]
    sg = [sg0, sg1]
    so = [so0, so1]
    iota = lax.iota(jnp.int32, LANES)

    pltpu.sync_copy(posT_hbm, posT_v)

    def pos_init(s, _):
        for c in range(EMBED // LANES):
            pos_v[s, pl.ds(c * LANES, LANES)] = plsc.load_gather(
                posT_v, [iota + c * LANES, jnp.full((LANES,), s, jnp.int32)]
            )
        return ()

    lax.fori_loop(0, SEQ, pos_init, ())

    def shift_seq(par, bp, j):
        for q in range(8):
            sl = pl.ds(q * LANES, LANES)
            pairs[par][sl] = lax.shift_right_logical(idx8_v[bp, j, sl], 1)

    def gather_copy(par):
        return pltpu.make_async_copy(
            tok2_hbm.at[pairs[par]], rows[par], sg[par]
        )

    def idx_copy(p, bp):
        return pltpu.make_async_copy(
            idxT_hbm.at[p, :, pl.ds(b0, BPW)], idx8_v.at[bp], sem_i
        )

    def out_copy(s, par):
        return pltpu.make_async_copy(
            outs[par], outT_hbm.at[s, :, pl.ds(b0, BPW)], so[par]
        )

    def valu_pos(s, par, bp, j):
        pvs = [pos_v[s, pl.ds(c * LANES, LANES)] for c in range(EMBED // LANES)]

        def group_body(g, _):
            rvec = g * LANES + iota
            hv = (idx8_v[bp, j, pl.ds(g * LANES, LANES)] & 1) * EMBED
            for e in range(EMBED):
                col = hv + e
                pe = pvs[e // LANES][e % LANES]
                outs[par][e, pl.ds(g * LANES, LANES)] = (
                    plsc.load_gather(rows[par], [rvec, col]) + pe
                )
            return ()

        lax.fori_loop(0, BPW // LANES, group_body, ())

    pltpu.sync_copy(idxT_hbm.at[0, :, pl.ds(b0, BPW)], idx8_v.at[0])
    shift_seq(0, 0, 0)
    gather_copy(0).start()

    def block_body(p, _):
        bp = p & 1

        @pl.when(p < NBLK - 1)
        def _():
            idx_copy(p + 1, 1 - bp).start()

        for j in range(8):
            s = 8 * p + j
            par = j % 2
            if j < 7:
                shift_seq(1 - par, bp, j + 1)
                gather_copy(1 - par).start()
            else:
                @pl.when(p < NBLK - 1)
                def _():
                    idx_copy(p + 1, 1 - bp).wait()
                    shift_seq(1 - par, 1 - bp, 0)
                    gather_copy(1 - par).start()

            gather_copy(par).wait()

            @pl.when(s >= 2)
            def _():
                out_copy(s, par).wait()

            valu_pos(s, par, bp, j)
            out_copy(s, par).start()
        return ()

    lax.fori_loop(0, NBLK, block_body, ())

    out_copy(0, 0).wait()
    out_copy(0, 1).wait()


@jax.jit
def kernel(inputs, token_table, position_table):
    tok2 = token_table.reshape(VOCAB // 2, PAIR)
    idxT = inputs.T.reshape(NBLK, 8, BATCH)
    posT = position_table.T
    mesh = plsc.VectorSubcoreMesh(
        core_axis_name="c", subcore_axis_name="s", num_cores=NUM_CORES,
        num_subcores=NUM_SUBCORES,
    )
    outT = pl.kernel(
        _body,
        out_type=jax.ShapeDtypeStruct((SEQ, EMBED, BATCH), jnp.float32),
        mesh=mesh,
        scratch_types=[
            pltpu.VMEM((2, 8, BPW), jnp.int32),
            pltpu.VMEM((BPW,), jnp.int32),
            pltpu.VMEM((BPW,), jnp.int32),
            pltpu.VMEM((BPW, PAIR), jnp.float32),
            pltpu.VMEM((BPW, PAIR), jnp.float32),
            pltpu.VMEM((EMBED, BPW), jnp.float32),
            pltpu.VMEM((EMBED, BPW), jnp.float32),
            pltpu.VMEM((EMBED, SEQ), jnp.float32),
            pltpu.VMEM((SEQ, EMBED), jnp.float32),
            pltpu.SemaphoreType.DMA,
            pltpu.SemaphoreType.DMA,
            pltpu.SemaphoreType.DMA,
            pltpu.SemaphoreType.DMA,
            pltpu.SemaphoreType.DMA,
        ],
        compiler_params=pltpu.CompilerParams(needs_layout_passes=False),
    )(idxT, tok2, posT)
    return outT.transpose(2, 0, 1)
